# Initial kernel scaffold; baseline (speedup 1.0000x reference)
import jax
import jax.numpy as jnp
from jax.experimental import pallas as pl


def kernel(weekday, start_time, sem_O, lngs, lats, travel_dis, spd, azimuth, sem_pt, weekday_W, start_time_W, sem_pt_W, fc_W):
    raise NotImplementedError("write your pallas kernel here")


# speedup vs baseline: 2.2442x; 2.2442x over previous
import jax
import jax.numpy as jnp
from jax import lax
from jax.experimental import pallas as pl

B = 4096
L = 200
BB = 64


def _body(wd_ref, st_ref, semO_ref, lngs_ref, lats_ref, td_ref, spd_ref,
          az_ref, spt_ref, wdW_ref, stW_ref, sptW_ref, fcW_ref,
          out_ref, traj_ref):
    bB = lngs_ref.shape[0]
    l = lngs_ref.shape[1]

    wd_oh = (wd_ref[...] == lax.broadcasted_iota(jnp.int32, (bB, 7), 1)
             ).astype(jnp.float32)
    wd = jnp.dot(wd_oh, wdW_ref[...], preferred_element_type=jnp.float32)
    st_oh = (st_ref[...] == lax.broadcasted_iota(jnp.int32, (bB, 48), 1)
             ).astype(jnp.float32)
    st = jnp.dot(st_oh, stW_ref[...], preferred_element_type=jnp.float32)
    sem = lax.dot_general(semO_ref[...], fcW_ref[...],
                          (((1,), (1,)), ((), ())),
                          preferred_element_type=jnp.float32)
    traj = jnp.concatenate([wd, st, sem], axis=1)
    traj_ref[...] = traj

    spt = spt_ref[...]
    embs = []
    for c in range(3):
        acc = jnp.zeros((bB, l), jnp.float32)
        for k in range(1, 9):
            acc = jnp.where(spt == k, sptW_ref[k, c], acc)
        embs.append(acc[..., None])

    out_ref[:, :, 0:1] = lngs_ref[...][..., None]
    out_ref[:, :, 1:2] = lats_ref[...][..., None]
    out_ref[:, :, 2:3] = td_ref[...][..., None]
    out_ref[:, :, 3:4] = spd_ref[...][..., None]
    out_ref[:, :, 4:5] = az_ref[...][..., None]
    out_ref[:, :, 5:17] = jnp.broadcast_to(traj[:, None, :], (bB, l, 12))
    out_ref[:, :, 17:18] = embs[0]
    out_ref[:, :, 18:19] = embs[1]
    out_ref[:, :, 19:20] = embs[2]


@jax.jit
def kernel(weekday, start_time, sem_O, lngs, lats, travel_dis, spd, azimuth,
           sem_pt, weekday_W, start_time_W, sem_pt_W, fc_W):
    wd2 = weekday.astype(jnp.int32).reshape(B, 1)
    st2 = start_time.astype(jnp.int32).reshape(B, 1)
    grid = (B // BB,)
    row = lambda i: (i, 0)
    full2 = lambda i: (0, 0)
    out, traj = pl.pallas_call(
        _body,
        grid=grid,
        in_specs=[
            pl.BlockSpec((BB, 1), row),
            pl.BlockSpec((BB, 1), row),
            pl.BlockSpec((BB, 8), row),
            pl.BlockSpec((BB, L), row),
            pl.BlockSpec((BB, L), row),
            pl.BlockSpec((BB, L), row),
            pl.BlockSpec((BB, L), row),
            pl.BlockSpec((BB, L), row),
            pl.BlockSpec((BB, L), row),
            pl.BlockSpec((7, 3), full2),
            pl.BlockSpec((48, 6), full2),
            pl.BlockSpec((9, 3), full2),
            pl.BlockSpec((3, 8), full2),
        ],
        out_specs=[
            pl.BlockSpec((BB, L, 20), lambda i: (i, 0, 0)),
            pl.BlockSpec((BB, 12), row),
        ],
        out_shape=[
            jax.ShapeDtypeStruct((B, L, 20), jnp.float32),
            jax.ShapeDtypeStruct((B, 12), jnp.float32),
        ],
    )(wd2, st2, sem_O, lngs, lats, travel_dis, spd, azimuth,
      sem_pt.astype(jnp.int32), weekday_W, start_time_W, sem_pt_W, fc_W)
    return (out, traj)
